# baseline (speedup 1.0000x reference)
import functools

import jax
import jax.numpy as jnp
from jax import lax
from jax.experimental import pallas as pl
from jax.experimental.pallas import tpu as pltpu
from jax.experimental.pallas import tpu_sc as plsc

H = 64
HP = 128
BASKET = 20
NC, NS = 2, 16
NW = NC * NS
LANES = 16
PACK_BLK = 8192
PACK_SH = 13


def _derive(raw, pair, col, n):
    for i in range(n // LANES):
        v = raw[pl.ds(i * LANES, LANES)]
        q = v >> PACK_SH
        col[pl.ds(i * LANES, LANES)] = (q & 1) << 6
        pair[pl.ds(i * LANES, LANES)] = ((q >> 1) << PACK_SH) | (
            v & (PACK_BLK - 1))


def _sc_gather_kernel(batch):
    b_per_w = batch // NW
    n_chunks = 16
    rpc = b_per_w // n_chunks
    s_chunk = rpc * BASKET

    mesh = plsc.VectorSubcoreMesh(
        core_axis_name="c", subcore_axis_name="s",
        num_cores=NC, num_subcores=NS)

    @functools.partial(
        pl.kernel,
        out_type=(
            jax.ShapeDtypeStruct((batch, H), jnp.float32),
            jax.ShapeDtypeStruct((batch, H), jnp.float32),
        ),
        mesh=mesh,
        scratch_types=dict(
            u_raw=pltpu.VMEM((b_per_w,), jnp.int32),
            u_pair=pltpu.VMEM((b_per_w,), jnp.int32),
            u_col=pltpu.VMEM((b_per_w + LANES,), jnp.int32),
            a_raw=pltpu.VMEM((b_per_w,), jnp.int32),
            a_pair=pltpu.VMEM((b_per_w,), jnp.int32),
            a_col=pltpu.VMEM((b_per_w + LANES,), jnp.int32),
            s_raw=[pltpu.VMEM((s_chunk,), jnp.int32)] * 2,
            s_pair=[pltpu.VMEM((s_chunk,), jnp.int32)] * 2,
            s_col=[pltpu.VMEM((s_chunk + LANES,), jnp.int32)] * 2,
            s_rows=[pltpu.VMEM((s_chunk, HP), jnp.float32)] * 2,
            usr_rows=pltpu.VMEM((b_per_w, HP), jnp.float32),
            a_rows=pltpu.VMEM((b_per_w, HP), jnp.float32),
            lhs_buf=pltpu.VMEM((b_per_w, H), jnp.float32),
            rhs_buf=pltpu.VMEM((b_per_w, H), jnp.float32),
            sem_u=pltpu.SemaphoreType.DMA,
            sem_a=pltpu.SemaphoreType.DMA,
            sem_raw=[pltpu.SemaphoreType.DMA] * 2,
            sem_s=[pltpu.SemaphoreType.DMA] * 2,
        ),
        compiler_params=pltpu.CompilerParams(use_tc_tiling_on_sc=True),
    )
    def sc_fn(u_hbm, a_hbm, s_hbm, item_hbm, usr_hbm, lhs_hbm, rhs_hbm,
              u_raw, u_pair, u_col, a_raw, a_pair, a_col,
              s_raw, s_pair, s_col, s_rows,
              usr_rows, a_rows, lhs_buf, rhs_buf,
              sem_u, sem_a, sem_raw, sem_s):
        wid = lax.axis_index("s") * NC + lax.axis_index("c")
        base = wid * b_per_w
        sbase = base * BASKET

        raws = [None, None]
        raws[0] = pltpu.async_copy(
            s_hbm.at[pl.ds(sbase, s_chunk)], s_raw[0], sem_raw[0])
        cp_ur = pltpu.async_copy(u_hbm.at[pl.ds(base, b_per_w)], u_raw, sem_u)
        cp_ar = pltpu.async_copy(a_hbm.at[pl.ds(base, b_per_w)], a_raw, sem_a)
        raws[0].wait()
        _derive(s_raw[0], s_pair[0], s_col[0], s_chunk)
        gathers = [None, None]
        gathers[0] = pltpu.async_copy(
            item_hbm.at[s_pair[0]], s_rows[0], sem_s[0])
        if n_chunks > 1:
            raws[1] = pltpu.async_copy(
                s_hbm.at[pl.ds(sbase + s_chunk, s_chunk)], s_raw[1],
                sem_raw[1])
        cp_ur.wait()
        _derive(u_raw, u_pair, u_col, b_per_w)
        cp_u = pltpu.async_copy(usr_hbm.at[u_pair], usr_rows, sem_u)
        cp_ar.wait()
        _derive(a_raw, a_pair, a_col, b_per_w)
        cp_a = pltpu.async_copy(item_hbm.at[a_pair], a_rows, sem_a)
        cp_u.wait()

        for c in range(n_chunks):
            b = c % 2
            nb = (c + 1) % 2
            if c + 1 < n_chunks:
                raws[nb].wait()
                _derive(s_raw[nb], s_pair[nb], s_col[nb], s_chunk)
                gathers[nb] = pltpu.async_copy(
                    item_hbm.at[s_pair[nb]], s_rows[nb], sem_s[nb])
            if c + 2 < n_chunks:
                raws[b] = pltpu.async_copy(
                    s_hbm.at[pl.ds(sbase + (c + 2) * s_chunk, s_chunk)],
                    s_raw[b], sem_raw[b])
            gathers[b].wait()

            rows_v, col_v = s_rows[b], s_col[b]

            def body(r, _, rows_v=rows_v, col_v=col_v, c=c):
                row = c * rpc + r
                ucol = u_col[pl.ds(row, LANES)][0]
                acc0 = usr_rows[row, pl.ds(ucol, LANES)]
                acc1 = usr_rows[row, pl.ds(ucol + LANES, LANES)]
                acc2 = usr_rows[row, pl.ds(ucol + 2 * LANES, LANES)]
                acc3 = usr_rows[row, pl.ds(ucol + 3 * LANES, LANES)]
                for j in range(BASKET):
                    p = r * BASKET + j
                    col = col_v[pl.ds(p, LANES)][0]
                    acc0 = acc0 + rows_v[p, pl.ds(col, LANES)]
                    acc1 = acc1 + rows_v[p, pl.ds(col + LANES, LANES)]
                    acc2 = acc2 + rows_v[p, pl.ds(col + 2 * LANES, LANES)]
                    acc3 = acc3 + rows_v[p, pl.ds(col + 3 * LANES, LANES)]
                lhs_buf[row, pl.ds(0, LANES)] = acc0
                lhs_buf[row, pl.ds(LANES, LANES)] = acc1
                lhs_buf[row, pl.ds(2 * LANES, LANES)] = acc2
                lhs_buf[row, pl.ds(3 * LANES, LANES)] = acc3
                return _

            lax.fori_loop(0, rpc, body, None)

        cp_a.wait()

        def a_body(r, _):
            col = a_col[pl.ds(r, LANES)][0]
            for h in range(H // LANES):
                rhs_buf[r, pl.ds(h * LANES, LANES)] = (
                    a_rows[r, pl.ds(col + h * LANES, LANES)])
            return _

        lax.fori_loop(0, b_per_w, a_body, None)

        pltpu.sync_copy(lhs_buf, lhs_hbm.at[pl.ds(base, b_per_w)])
        pltpu.sync_copy(rhs_buf, rhs_hbm.at[pl.ds(base, b_per_w)])

    return sc_fn


def _pack_body(lo_ref, hi_ref, out_ref):
    x = jnp.concatenate([lo_ref[...], hi_ref[...]], axis=0)
    out_ref[...] = x.T


def _tc_pack(table_t, blk=PACK_BLK):
    v = table_t.shape[1]
    n_pairs = -(-v // (2 * blk))
    rows = n_pairs * blk
    last = -(-v // blk) - 1
    return pl.pallas_call(
        _pack_body,
        out_shape=jax.ShapeDtypeStruct((rows, HP), jnp.float32),
        grid=(n_pairs,),
        in_specs=[
            pl.BlockSpec((H, blk), lambda k: (0, jnp.minimum(2 * k, last))),
            pl.BlockSpec(
                (H, blk), lambda k: (0, jnp.minimum(2 * k + 1, last))),
        ],
        out_specs=pl.BlockSpec((blk, HP), lambda k: (k, 0)),
    )(table_t, table_t)


def _mm_body(lhs_ref, rhs_ref, out_ref):
    out_ref[...] = lax.dot_general(
        lhs_ref[...], rhs_ref[...],
        dimension_numbers=(((1,), (1,)), ((), ())),
        preferred_element_type=jnp.float32,
    )


def _tc_matmul(lhs, rhs, blk_m=2048, blk_n=2048):
    batch = lhs.shape[0]
    grid = (batch // blk_m, batch // blk_n)
    return pl.pallas_call(
        _mm_body,
        out_shape=jax.ShapeDtypeStruct((batch, batch), jnp.float32),
        grid=grid,
        in_specs=[
            pl.BlockSpec((blk_m, H), lambda i, j: (i, 0)),
            pl.BlockSpec((blk_n, H), lambda i, j: (j, 0)),
        ],
        out_specs=pl.BlockSpec((blk_m, blk_n), lambda i, j: (i, j)),
    )(lhs, rhs)


@jax.jit
def kernel(U, S, A, B, item_embedding, usr_embedding):
    batch = U.shape[0]
    item2 = _tc_pack(item_embedding.T)
    usr2 = _tc_pack(usr_embedding.T)
    s_last = S[:, -1, :].reshape(-1).astype(jnp.int32)
    lhs, rhs = _sc_gather_kernel(batch)(
        U.astype(jnp.int32), A.astype(jnp.int32), s_last, item2, usr2)
    return _tc_matmul(lhs, rhs)


# speedup vs baseline: 1.0134x; 1.0134x over previous
import functools

import jax
import jax.numpy as jnp
from jax import lax
from jax.experimental import pallas as pl
from jax.experimental.pallas import tpu as pltpu
from jax.experimental.pallas import tpu_sc as plsc

H = 64
HP = 128
BASKET = 20
NC, NS = 2, 16
NW = NC * NS
LANES = 16
PACK_BLK = 16384
PACK_SH = 14


def _derive(raw, pair, col, n):
    for i in range(n // LANES):
        v = raw[pl.ds(i * LANES, LANES)]
        q = v >> PACK_SH
        col[pl.ds(i * LANES, LANES)] = (q & 1) << 6
        pair[pl.ds(i * LANES, LANES)] = ((q >> 1) << PACK_SH) | (
            v & (PACK_BLK - 1))


def _sc_gather_kernel(batch):
    b_per_w = batch // NW
    n_chunks = 16
    rpc = b_per_w // n_chunks
    s_chunk = rpc * BASKET

    mesh = plsc.VectorSubcoreMesh(
        core_axis_name="c", subcore_axis_name="s",
        num_cores=NC, num_subcores=NS)

    @functools.partial(
        pl.kernel,
        out_type=(
            jax.ShapeDtypeStruct((batch, H), jnp.float32),
            jax.ShapeDtypeStruct((batch, H), jnp.float32),
        ),
        mesh=mesh,
        scratch_types=dict(
            u_raw=pltpu.VMEM((b_per_w,), jnp.int32),
            u_pair=pltpu.VMEM((b_per_w,), jnp.int32),
            u_col=pltpu.VMEM((b_per_w + LANES,), jnp.int32),
            a_raw=pltpu.VMEM((b_per_w,), jnp.int32),
            a_pair=pltpu.VMEM((b_per_w,), jnp.int32),
            a_col=pltpu.VMEM((b_per_w + LANES,), jnp.int32),
            s_raw=[pltpu.VMEM((s_chunk,), jnp.int32)] * 2,
            s_pair=[pltpu.VMEM((s_chunk,), jnp.int32)] * 2,
            s_col=[pltpu.VMEM((s_chunk + LANES,), jnp.int32)] * 2,
            s_rows=[pltpu.VMEM((s_chunk, HP), jnp.float32)] * 2,
            usr_rows=pltpu.VMEM((b_per_w, HP), jnp.float32),
            a_rows=pltpu.VMEM((b_per_w, HP), jnp.float32),
            lhs_buf=pltpu.VMEM((b_per_w, H), jnp.float32),
            rhs_buf=pltpu.VMEM((b_per_w, H), jnp.float32),
            sem_u=pltpu.SemaphoreType.DMA,
            sem_a=pltpu.SemaphoreType.DMA,
            sem_raw=[pltpu.SemaphoreType.DMA] * 2,
            sem_s=[pltpu.SemaphoreType.DMA] * 2,
        ),
        compiler_params=pltpu.CompilerParams(use_tc_tiling_on_sc=True),
    )
    def sc_fn(u_hbm, a_hbm, s_hbm, item_hbm, usr_hbm, lhs_hbm, rhs_hbm,
              u_raw, u_pair, u_col, a_raw, a_pair, a_col,
              s_raw, s_pair, s_col, s_rows,
              usr_rows, a_rows, lhs_buf, rhs_buf,
              sem_u, sem_a, sem_raw, sem_s):
        wid = lax.axis_index("s") * NC + lax.axis_index("c")
        base = wid * b_per_w
        sbase = base * BASKET

        raws = [None, None]
        raws[0] = pltpu.async_copy(
            s_hbm.at[pl.ds(sbase, s_chunk)], s_raw[0], sem_raw[0])
        cp_ur = pltpu.async_copy(u_hbm.at[pl.ds(base, b_per_w)], u_raw, sem_u)
        cp_ar = pltpu.async_copy(a_hbm.at[pl.ds(base, b_per_w)], a_raw, sem_a)
        raws[0].wait()
        _derive(s_raw[0], s_pair[0], s_col[0], s_chunk)
        gathers = [None, None]
        gathers[0] = pltpu.async_copy(
            item_hbm.at[s_pair[0]], s_rows[0], sem_s[0])
        if n_chunks > 1:
            raws[1] = pltpu.async_copy(
                s_hbm.at[pl.ds(sbase + s_chunk, s_chunk)], s_raw[1],
                sem_raw[1])
        cp_ur.wait()
        _derive(u_raw, u_pair, u_col, b_per_w)
        cp_u = pltpu.async_copy(usr_hbm.at[u_pair], usr_rows, sem_u)
        cp_ar.wait()
        _derive(a_raw, a_pair, a_col, b_per_w)
        cp_a = pltpu.async_copy(item_hbm.at[a_pair], a_rows, sem_a)
        cp_u.wait()

        for c in range(n_chunks):
            b = c % 2
            nb = (c + 1) % 2
            if c + 1 < n_chunks:
                raws[nb].wait()
                _derive(s_raw[nb], s_pair[nb], s_col[nb], s_chunk)
                gathers[nb] = pltpu.async_copy(
                    item_hbm.at[s_pair[nb]], s_rows[nb], sem_s[nb])
            if c + 2 < n_chunks:
                raws[b] = pltpu.async_copy(
                    s_hbm.at[pl.ds(sbase + (c + 2) * s_chunk, s_chunk)],
                    s_raw[b], sem_raw[b])
            gathers[b].wait()

            rows_v, col_v = s_rows[b], s_col[b]

            def body(r, _, rows_v=rows_v, col_v=col_v, c=c):
                row = c * rpc + r
                ucol = u_col[pl.ds(row, LANES)][0]
                acc0 = usr_rows[row, pl.ds(ucol, LANES)]
                acc1 = usr_rows[row, pl.ds(ucol + LANES, LANES)]
                acc2 = usr_rows[row, pl.ds(ucol + 2 * LANES, LANES)]
                acc3 = usr_rows[row, pl.ds(ucol + 3 * LANES, LANES)]
                for j in range(BASKET):
                    p = r * BASKET + j
                    col = col_v[pl.ds(p, LANES)][0]
                    acc0 = acc0 + rows_v[p, pl.ds(col, LANES)]
                    acc1 = acc1 + rows_v[p, pl.ds(col + LANES, LANES)]
                    acc2 = acc2 + rows_v[p, pl.ds(col + 2 * LANES, LANES)]
                    acc3 = acc3 + rows_v[p, pl.ds(col + 3 * LANES, LANES)]
                lhs_buf[row, pl.ds(0, LANES)] = acc0
                lhs_buf[row, pl.ds(LANES, LANES)] = acc1
                lhs_buf[row, pl.ds(2 * LANES, LANES)] = acc2
                lhs_buf[row, pl.ds(3 * LANES, LANES)] = acc3
                return _

            lax.fori_loop(0, rpc, body, None)

        cp_a.wait()

        def a_body(r, _):
            col = a_col[pl.ds(r, LANES)][0]
            for h in range(H // LANES):
                rhs_buf[r, pl.ds(h * LANES, LANES)] = (
                    a_rows[r, pl.ds(col + h * LANES, LANES)])
            return _

        lax.fori_loop(0, b_per_w, a_body, None)

        pltpu.sync_copy(lhs_buf, lhs_hbm.at[pl.ds(base, b_per_w)])
        pltpu.sync_copy(rhs_buf, rhs_hbm.at[pl.ds(base, b_per_w)])

    return sc_fn


def _pack_body(lo_ref, hi_ref, out_ref):
    x = jnp.concatenate([lo_ref[...], hi_ref[...]], axis=0)
    out_ref[...] = x.T


def _tc_pack(table_t, blk=PACK_BLK):
    v = table_t.shape[1]
    n_pairs = -(-v // (2 * blk))
    rows = n_pairs * blk
    last = -(-v // blk) - 1
    return pl.pallas_call(
        _pack_body,
        out_shape=jax.ShapeDtypeStruct((rows, HP), jnp.float32),
        grid=(n_pairs,),
        in_specs=[
            pl.BlockSpec((H, blk), lambda k: (0, jnp.minimum(2 * k, last))),
            pl.BlockSpec(
                (H, blk), lambda k: (0, jnp.minimum(2 * k + 1, last))),
        ],
        out_specs=pl.BlockSpec((blk, HP), lambda k: (k, 0)),
    )(table_t, table_t)


def _mm_body(lhs_ref, rhs_ref, out_ref):
    out_ref[...] = lax.dot_general(
        lhs_ref[...], rhs_ref[...],
        dimension_numbers=(((1,), (1,)), ((), ())),
        preferred_element_type=jnp.float32,
    )


def _tc_matmul(lhs, rhs, blk_m=2048, blk_n=2048):
    batch = lhs.shape[0]
    grid = (batch // blk_m, batch // blk_n)
    return pl.pallas_call(
        _mm_body,
        out_shape=jax.ShapeDtypeStruct((batch, batch), jnp.float32),
        grid=grid,
        in_specs=[
            pl.BlockSpec((blk_m, H), lambda i, j: (i, 0)),
            pl.BlockSpec((blk_n, H), lambda i, j: (j, 0)),
        ],
        out_specs=pl.BlockSpec((blk_m, blk_n), lambda i, j: (i, j)),
    )(lhs, rhs)


@jax.jit
def kernel(U, S, A, B, item_embedding, usr_embedding):
    batch = U.shape[0]
    item2 = _tc_pack(item_embedding.T)
    usr2 = _tc_pack(usr_embedding.T)
    s_last = S[:, -1, :].reshape(-1).astype(jnp.int32)
    lhs, rhs = _sc_gather_kernel(batch)(
        U.astype(jnp.int32), A.astype(jnp.int32), s_last, item2, usr2)
    return _tc_matmul(lhs, rhs)
